# baseline (speedup 1.0000x reference)
import functools

import jax
import jax.numpy as jnp
from jax.experimental import pallas as pl
from jax.experimental.pallas import tpu as pltpu
from jax.experimental.pallas import tpu_sc as plsc

_PAIRS = [[0,3],[1,17],[0,250],[1,999],[0,1500],[1,4096],[0,7777],[1,12345],[0,20000],[1,33333],[0,45000],[1,54321],[0,60000],[1,77777],[0,88888],[1,99998],[1,3],[0,17],[1,250],[0,999],[1,1500],[0,4096],[1,7777],[0,12345],[1,20000],[0,33333],[1,45000],[0,54321],[1,60000],[0,77777],[1,88888],[0,99998],[0,3],[0,99998],[1,17],[1,88888],[0,250],[0,77777],[1,999],[1,60000],[0,1500],[0,54321],[1,4096],[1,45000],[0,7777],[0,33333],[1,12345],[1,20000]]

_N_OUT = len(_PAIRS)
_D = 128


def _sc_body(t0_hbm, t1_hbm, out_hbm, sem):
    tables = (t0_hbm, t1_hbm)
    for i, (layer, row) in enumerate(_PAIRS):
        pltpu.async_copy(tables[layer].at[row], out_hbm.at[i], sem)
    pltpu.make_async_copy(t0_hbm.at[pl.ds(0, _N_OUT)], out_hbm, sem).wait()


_gather_call = functools.partial(
    pl.kernel,
    mesh=plsc.ScalarSubcoreMesh(axis_name="c", num_cores=1),
    out_type=jax.ShapeDtypeStruct((_N_OUT, _D), jnp.float32),
    scratch_types=[
        pltpu.SemaphoreType.DMA,
    ],
)(_sc_body)


@jax.jit
def kernel(layer_values_0, layer_values_1):
    return _gather_call(layer_values_0, layer_values_1)
